# baseline (speedup 1.0000x reference)
import functools

import jax
import jax.numpy as jnp
from jax import lax
from jax.experimental import pallas as pl
from jax.experimental.pallas import tpu as pltpu
from jax.experimental.pallas import tpu_sc as plsc

_L = 16


def _sc_body(x_hbm, emb_hbm, o_hbm, *, nseq, batch, br, d):
    def block_body(x_v, emb_v, o_v):
        @pl.loop(0, br)
        def _row(r):
            @pl.loop(0, d, step=_L)
            def _col(c):
                o_v.at[r, pl.ds(c, _L)][...] = (
                    x_v.at[r, pl.ds(c, _L)][...] + emb_v.at[r, pl.ds(c, _L)][...]
                )

    pltpu.emit_pipeline(
        block_body,
        grid=(nseq, batch),
        in_specs=[
            pl.BlockSpec((br, d), index_map=lambda i, b: (b * nseq + i, 0)),
            pl.BlockSpec((br, d), index_map=lambda i, b: (i, 0)),
        ],
        out_specs=[pl.BlockSpec((br, d), index_map=lambda i, b: (b * nseq + i, 0))],
        core_axis_name=("c", "s"),
        dimension_semantics=(pltpu.PARALLEL, pltpu.ARBITRARY),
    )(x_hbm, emb_hbm, o_hbm)


def kernel(x, emb):
    B, T, D = x.shape
    BR = 16
    nseq = T // BR
    x2 = x.reshape(B * T, D)
    mesh = plsc.VectorSubcoreMesh(core_axis_name="c", subcore_axis_name="s")
    body = functools.partial(_sc_body, nseq=nseq, batch=B, br=BR, d=D)
    run = pl.kernel(
        body,
        out_type=jax.ShapeDtypeStruct((B * T, D), x.dtype),
        mesh=mesh,
    )
    return run(x2, emb).reshape(B, T, D)


# speedup vs baseline: 1.1091x; 1.1091x over previous
import functools

import jax
import jax.numpy as jnp
from jax import lax
from jax.experimental import pallas as pl
from jax.experimental.pallas import tpu as pltpu
from jax.experimental.pallas import tpu_sc as plsc

_L = 16


def _sc_body(x_hbm, emb_hbm, o_hbm, *, nseq, batch, br, d):
    def block_body(x_v, emb_v, o_v):
        @pl.loop(0, br)
        def _row(r):
            for c in range(0, d, _L):
                o_v.at[r, pl.ds(c, _L)][...] = (
                    x_v.at[r, pl.ds(c, _L)][...] + emb_v.at[r, pl.ds(c, _L)][...]
                )

    pltpu.emit_pipeline(
        block_body,
        grid=(nseq, batch),
        in_specs=[
            pl.BlockSpec((br, d), index_map=lambda i, b: (b * nseq + i, 0)),
            pl.BlockSpec((br, d), index_map=lambda i, b: (i, 0)),
        ],
        out_specs=[pl.BlockSpec((br, d), index_map=lambda i, b: (b * nseq + i, 0))],
        core_axis_name=("c", "s"),
        dimension_semantics=(pltpu.PARALLEL, pltpu.ARBITRARY),
    )(x_hbm, emb_hbm, o_hbm)


def kernel(x, emb):
    B, T, D = x.shape
    BR = 16
    nseq = T // BR
    x2 = x.reshape(B * T, D)
    mesh = plsc.VectorSubcoreMesh(core_axis_name="c", subcore_axis_name="s")
    body = functools.partial(_sc_body, nseq=nseq, batch=B, br=BR, d=D)
    run = pl.kernel(
        body,
        out_type=jax.ShapeDtypeStruct((B * T, D), x.dtype),
        mesh=mesh,
    )
    return run(x2, emb).reshape(B, T, D)
